# baseline (speedup 1.0000x reference)
import functools

import jax
import jax.numpy as jnp
from jax import lax
from jax.experimental import pallas as pl
from jax.experimental.pallas import tpu as pltpu
from jax.experimental.pallas import tpu_sc as plsc

EMB = 128
NC, NS = 2, 16
NW = NC * NS


def _table_body(w0_ref, w1_ref, w2_ref, t_ref):
    c = lax.broadcasted_iota(jnp.int32, (64, 1), 0)
    i0, r = c // 12, c % 12
    i1, i2 = r // 2, r % 2

    def pick(idx, w_ref):
        acc = jnp.zeros((64, EMB), jnp.float32)
        for row in range(w_ref.shape[0]):
            acc = acc + (idx == row).astype(jnp.float32) * w_ref[row:row + 1, :]
        return acc

    t_ref[...] = (pick(i0, w0_ref) + pick(i1, w1_ref) + pick(i2, w2_ref))


def _build_table(W0, W1, W2):
    def pad8(w):
        return jnp.zeros((8, EMB), jnp.float32).at[:w.shape[0]].set(w)

    return pl.pallas_call(
        _table_body,
        out_shape=jax.ShapeDtypeStruct((64, EMB), jnp.float32),
    )(pad8(W0), pad8(W1), pad8(W2))


def _make_sc_kernel(E):
    per_w = E // NW
    chunk = 200
    sub = 40
    nsub = chunk // sub
    n_groups = per_w // (2 * chunk)
    assert E % (NW * 2 * chunk) == 0
    mesh = plsc.VectorSubcoreMesh(core_axis_name="c", subcore_axis_name="s")

    @functools.partial(
        pl.kernel, mesh=mesh,
        out_type=jax.ShapeDtypeStruct((E, EMB), jnp.float32),
        scratch_types=[
            pltpu.VMEM_SHARED((64, EMB), jnp.float32),
            pltpu.VMEM((per_w,), jnp.int32),
            pltpu.VMEM((per_w,), jnp.int32),
            pltpu.VMEM((per_w,), jnp.int32),
            pltpu.VMEM((per_w,), jnp.int32),
            pltpu.VMEM((chunk, EMB), jnp.float32),
            pltpu.VMEM((chunk, EMB), jnp.float32),
            pltpu.SemaphoreType.DMA,
            pltpu.SemaphoreType.DMA,
            pltpu.SemaphoreType.DMA,
            pltpu.SemaphoreType.DMA,
        ])
    def k(attr_hbm, t_hbm, out_hbm, t_s, a0_v, a1_v, a2_v, codes_v, rows0,
          rows1, sem_g0, sem_g1, sem_o0, sem_o1):
        wid = lax.axis_index("s") * NC + lax.axis_index("c")
        tile_base = wid * per_w
        @pl.when(lax.axis_index("s") == 0)
        def _():
            pltpu.sync_copy(t_hbm, t_s)
        plsc.subcore_barrier()

        hs_a = [pltpu.async_copy(
                    attr_hbm.at[pl.ds(c * E + tile_base, per_w)], av, sem_g0)
                for c, av in enumerate((a0_v, a1_v, a2_v))]
        for h in hs_a:
            h.wait()

        rows = (rows0, rows1)
        sems_o = (sem_o0, sem_o1)
        sems_g = (sem_g0, sem_g1)

        def chunk_body(g, carry):
            pair_base = 2 * g * chunk
            for j in range(2 * chunk // 16):
                s = pl.ds(pair_base + j * 16, 16)
                codes_v[s] = a0_v[s] * 12 + a1_v[s] * 2 + a2_v[s]
            for b in range(2):
                base = (2 * g + b) * chunk
                rb, so = rows[b], sems_o[b]

                @pl.when(g >= 1)
                def _():
                    pltpu.make_async_copy(
                        rb, out_hbm.at[pl.ds(tile_base + base, chunk)],
                        so).wait()

                for j in range(nsub):
                    pltpu.async_copy(
                        t_s.at[codes_v.at[pl.ds(base + j * sub, sub)]],
                        rb.at[pl.ds(j * sub, sub)], sems_g[b])
            for b in range(2):
                base = (2 * g + b) * chunk
                rb, so = rows[b], sems_o[b]
                for j in range(nsub):
                    pltpu.make_async_copy(
                        t_s.at[codes_v.at[pl.ds(base + j * sub, sub)]],
                        rb.at[pl.ds(j * sub, sub)], sems_g[b]).wait()
                pltpu.async_copy(
                    rb, out_hbm.at[pl.ds(tile_base + base, chunk)], so)
            return carry

        lax.fori_loop(0, n_groups, chunk_body, 0)

        for b in range(2):
            base = (2 * (n_groups - 1) + b) * chunk
            pltpu.make_async_copy(
                rows[b], out_hbm.at[pl.ds(tile_base + base, chunk)],
                sems_o[b]).wait()

    return k


def kernel(edge_attr, W0, W1, W2):
    E = edge_attr.shape[0]
    attr = edge_attr.astype(jnp.int32).T.reshape(-1)
    t = _build_table(W0, W1, W2)
    return _make_sc_kernel(E)(attr, t)


# speedup vs baseline: 1.3145x; 1.3145x over previous
import functools

import jax
import jax.numpy as jnp
from jax import lax
from jax.experimental import pallas as pl
from jax.experimental.pallas import tpu as pltpu
from jax.experimental.pallas import tpu_sc as plsc

EMB = 128
NC, NS = 2, 16
NW = NC * NS


def _table_body(w0_ref, w1_ref, w2_ref, t_ref):
    c = lax.broadcasted_iota(jnp.int32, (64, 1), 0)
    i0, r = c // 12, c % 12
    i1, i2 = r // 2, r % 2

    def pick(idx, w_ref):
        acc = jnp.zeros((64, EMB), jnp.float32)
        for row in range(w_ref.shape[0]):
            acc = acc + (idx == row).astype(jnp.float32) * w_ref[row:row + 1, :]
        return acc

    t_ref[...] = (pick(i0, w0_ref) + pick(i1, w1_ref) + pick(i2, w2_ref))


def _build_table(W0, W1, W2):
    def pad8(w):
        return jnp.zeros((8, EMB), jnp.float32).at[:w.shape[0]].set(w)

    return pl.pallas_call(
        _table_body,
        out_shape=jax.ShapeDtypeStruct((64, EMB), jnp.float32),
    )(pad8(W0), pad8(W1), pad8(W2))


def _make_sc_kernel(E):
    per_w = E // NW
    chunk = 200
    sub = 40
    nsub = chunk // sub
    n_groups = per_w // (2 * chunk)
    assert E % (NW * 2 * chunk) == 0
    mesh = plsc.VectorSubcoreMesh(core_axis_name="c", subcore_axis_name="s")

    @functools.partial(
        pl.kernel, mesh=mesh,
        out_type=jax.ShapeDtypeStruct((E, EMB), jnp.float32),
        scratch_types=[
            pltpu.VMEM_SHARED((64, EMB), jnp.float32),
            pltpu.VMEM((per_w,), jnp.int32),
            pltpu.VMEM((per_w,), jnp.int32),
            pltpu.VMEM((per_w,), jnp.int32),
            pltpu.VMEM((per_w,), jnp.int32),
            pltpu.VMEM((chunk, EMB), jnp.float32),
            pltpu.VMEM((chunk, EMB), jnp.float32),
            pltpu.SemaphoreType.DMA,
            pltpu.SemaphoreType.DMA,
            pltpu.SemaphoreType.DMA,
        ])
    def k(attr_hbm, t_hbm, out_hbm, t_s, a0_v, a1_v, a2_v, codes_v, rows0,
          rows1, sem_g, sem_o0, sem_o1):
        wid = lax.axis_index("s") * NC + lax.axis_index("c")
        tile_base = wid * per_w
        @pl.when(lax.axis_index("s") == 0)
        def _():
            pltpu.sync_copy(t_hbm, t_s)
        plsc.subcore_barrier()

        hs_a = [pltpu.async_copy(
                    attr_hbm.at[pl.ds(c * E + tile_base, per_w)], av, sem_g)
                for c, av in enumerate((a0_v, a1_v, a2_v))]
        for h in hs_a:
            h.wait()

        rows = (rows0, rows1)
        sems_o = (sem_o0, sem_o1)

        def chunk_body(g, carry):
            pair_base = 2 * g * chunk
            for j in range(2 * chunk // 16):
                s = pl.ds(pair_base + j * 16, 16)
                codes_v[s] = a0_v[s] * 12 + a1_v[s] * 2 + a2_v[s]
            for b in range(2):
                base = (2 * g + b) * chunk
                rb, so = rows[b], sems_o[b]

                @pl.when(g >= 1)
                def _():
                    pltpu.make_async_copy(
                        rb, out_hbm.at[pl.ds(tile_base + base, chunk)],
                        so).wait()

                hs = [pltpu.async_copy(
                          t_s.at[codes_v.at[pl.ds(base + j * sub, sub)]],
                          rb.at[pl.ds(j * sub, sub)], sem_g)
                      for j in range(nsub)]
                for h in hs:
                    h.wait()
                pltpu.async_copy(
                    rb, out_hbm.at[pl.ds(tile_base + base, chunk)], so)
            return carry

        lax.fori_loop(0, n_groups, chunk_body, 0)

        for b in range(2):
            base = (2 * (n_groups - 1) + b) * chunk
            pltpu.make_async_copy(
                rows[b], out_hbm.at[pl.ds(tile_base + base, chunk)],
                sems_o[b]).wait()

    return k


def kernel(edge_attr, W0, W1, W2):
    E = edge_attr.shape[0]
    attr = edge_attr.astype(jnp.int32).T.reshape(-1)
    t = _build_table(W0, W1, W2)
    return _make_sc_kernel(E)(attr, t)
